# baseline (speedup 1.0000x reference)
import functools

import jax
import jax.numpy as jnp
from jax.experimental import pallas as pl
from jax.experimental.pallas import tpu as pltpu

N, D, E, H_R = 4096, 1024, 16, 64
TBLK = 2048


def _moe_kernel(x_ref, rw1_ref, rb1_ref, rw2_ref, rb2_ref,
                ew1_ref, eb1_ref, ew2_ref, eb2_ref,
                y_ref, w_ref, wtop_ref):
    e = pl.program_id(1)

    @pl.when(e == 0)
    def _router():
        xb = x_ref[...]
        hr = jnp.maximum(
            jnp.dot(xb, rw1_ref[...], preferred_element_type=jnp.float32)
            + rb1_ref[...][None, :], 0.0)
        logits = (jnp.dot(hr, rw2_ref[...], preferred_element_type=jnp.float32)
                  + rb2_ref[...][None, :])
        logits = logits - jnp.max(logits, axis=-1, keepdims=True)
        ew = jnp.exp(logits)
        w = ew / jnp.sum(ew, axis=-1, keepdims=True)
        w_ref[...] = w
        cols = jax.lax.broadcasted_iota(jnp.int32, w.shape, 1)
        i1 = jnp.argmax(w, axis=-1)[:, None]
        w2 = jnp.where(cols == i1, -jnp.inf, w)
        i2 = jnp.argmax(w2, axis=-1)[:, None]
        mask = (cols == i1) | (cols == i2)
        wt = jnp.where(mask, w, 0.0)
        wtop_ref[...] = wt / (jnp.sum(wt, axis=-1, keepdims=True) + 1e-8)

    xb = x_ref[...]
    h = jnp.tanh(
        jnp.dot(xb, ew1_ref[0], preferred_element_type=jnp.float32)
        + eb1_ref[0])
    out = (jnp.dot(h, ew2_ref[0], preferred_element_type=jnp.float32)
           + eb2_ref[0])
    wt = wtop_ref[...]
    ecols = jax.lax.broadcasted_iota(jnp.int32, wt.shape, 1)
    wcol = jnp.sum(jnp.where(ecols == e, wt, 0.0), axis=-1, keepdims=True)
    contrib = wcol * out

    @pl.when(e == 0)
    def _init():
        y_ref[...] = contrib

    @pl.when(e != 0)
    def _acc():
        y_ref[...] += contrib


@jax.jit
def kernel(x, rw1, rb1, rw2, rb2, ew1, eb1, ew2, eb2):
    n_tblk = N // TBLK
    grid = (n_tblk, E)
    y, w = pl.pallas_call(
        _moe_kernel,
        grid=grid,
        in_specs=[
            pl.BlockSpec((TBLK, D), lambda i, e: (i, 0)),
            pl.BlockSpec((D, H_R), lambda i, e: (0, 0)),
            pl.BlockSpec((H_R,), lambda i, e: (0,)),
            pl.BlockSpec((H_R, E), lambda i, e: (0, 0)),
            pl.BlockSpec((E,), lambda i, e: (0,)),
            pl.BlockSpec((1, D, 128), lambda i, e: (e, 0, 0)),
            pl.BlockSpec((1, 1, 128), lambda i, e: (e, 0, 0)),
            pl.BlockSpec((1, 128, D), lambda i, e: (e, 0, 0)),
            pl.BlockSpec((1, 1, D), lambda i, e: (e, 0, 0)),
        ],
        out_specs=[
            pl.BlockSpec((TBLK, D), lambda i, e: (i, 0)),
            pl.BlockSpec((TBLK, E), lambda i, e: (i, 0)),
        ],
        out_shape=[
            jax.ShapeDtypeStruct((N, D), jnp.float32),
            jax.ShapeDtypeStruct((N, E), jnp.float32),
        ],
        scratch_shapes=[pltpu.VMEM((TBLK, E), jnp.float32)],
        compiler_params=pltpu.CompilerParams(
            dimension_semantics=("parallel", "arbitrary")),
    )(x, rw1, rb1, rw2, rb2, ew1, eb1[:, None, :], ew2, eb2[:, None, :])
    return (y, w)
